# Initial kernel scaffold; baseline (speedup 1.0000x reference)
import jax
import jax.numpy as jnp
from jax.experimental import pallas as pl


def kernel(feature):
    raise NotImplementedError("write your pallas kernel here")


# speedup vs baseline: 13.8261x; 13.8261x over previous
import jax
import jax.numpy as jnp
from jax.experimental import pallas as pl
from jax.experimental.pallas import tpu as pltpu

_GRP = 16
_CPG = 6
_NROW = _CPG * 9
_TOPK = 3


def _select_kernel(f_ref, s_ref):
    B, c, H, W = 2, _CPG, 224, 224
    Hh, Ww = H - 2, W - 2
    n = _NROW

    fq = f_ref[...]

    iy = jax.lax.broadcasted_iota(jnp.int32, (H, 256), 0)
    ix = jax.lax.broadcasted_iota(jnp.int32, (H, 256), 1)
    win = (iy < Hh) & (ix < Ww)

    iota_r = jax.lax.broadcasted_iota(jnp.int32, (n, n), 0)
    iota_c = jax.lax.broadcasted_iota(jnp.int32, (n, n), 1)

    idx_rows = []
    for b in range(B):
        pats = [fq[b, :, ki:ki + H, kj:kj + 256]
                for ki in range(3) for kj in range(3)]
        S = jnp.stack(pats, axis=1).reshape(n, H, 256)
        A = jnp.where(win[None], S, 0.0)
        A = A.reshape(n, H, 2, 128).reshape(n, 2 * H, 128)
        A = A.reshape(n, 2 * H * 128)
        Ab = A.astype(jnp.bfloat16)
        G = jax.lax.dot_general(
            Ab, Ab, (((1,), (1,)), ((), ())),
            preferred_element_type=jnp.float32)
        sq_r = jnp.sum(A * A, axis=1, keepdims=True)
        dsq = jnp.where(iota_r == iota_c, sq_r, 0.0)
        sq_c = jnp.sum(dsq, axis=0, keepdims=True)
        d2 = sq_r + sq_c - 2.0 * G
        dist = jnp.sqrt(jnp.maximum(d2, 0.0))
        dist = jnp.where(iota_r == iota_c, jnp.float32(jnp.inf), dist)
        mn = jnp.min(dist, axis=1, keepdims=True)
        fi = jnp.min(jnp.where(dist == mn, iota_c, n),
                     axis=1, keepdims=True)
        idx_rows.append(fi)
    idx_all = jnp.concatenate(idx_rows, axis=0)

    iota_cnt = jax.lax.broadcasted_iota(jnp.int32, (B * n, n), 1)
    counts = jnp.sum((idx_all == iota_cnt).astype(jnp.int32),
                     axis=0, keepdims=True)
    iota_54 = jax.lax.broadcasted_iota(jnp.int32, (1, n), 1)

    sel = []
    cwork = counts
    for _ in range(_TOPK):
        c_o = jnp.max(cwork)
        v_o = jnp.min(jnp.where(cwork == c_o, iota_54, n))
        sel.append((v_o, c_o))
        cwork = jnp.where(iota_54 == v_o, jnp.int32(-1), cwork)
    tot = sel[0][1] + sel[1][1] + sel[2][1]

    lane = jax.lax.broadcasted_iota(jnp.int32, (1, 8), 1)
    rec = jnp.where(lane == 0, sel[0][0],
          jnp.where(lane == 1, sel[1][0],
          jnp.where(lane == 2, sel[2][0],
          jnp.where(lane == 3, sel[0][1],
          jnp.where(lane == 4, sel[1][1],
          jnp.where(lane == 5, sel[2][1],
          jnp.where(lane == 6, tot, jnp.int32(0))))))))
    s_ref[...] = rec.reshape(1, 1, 8)


def _combine_kernel(f_ref, s_ref, o_ref, p_ref):
    B, c, H, W = 2, _CPG, 224, 224
    Hh, Ww = H - 2, W - 2

    fq = f_ref[...]
    tot = s_ref[0, 0, 6]

    oy = jax.lax.broadcasted_iota(jnp.int32, (H, W), 0)
    ox = jax.lax.broadcasted_iota(jnp.int32, (H, W), 1)

    p_ref[...] = jnp.zeros((H + 4, W + 4), jnp.float32)

    outs = []
    for b in range(B):
        fb = fq[b, :, :H, :W]
        accb = jnp.zeros((c, H, W), jnp.float32)
        for o in range(_TOPK):
            v_o = s_ref[0, 0, o]
            c_o = s_ref[0, 0, 3 + o]
            ch_o = v_o // 9
            ko = v_o - ch_o * 9
            kio = ko // 3
            kjo = ko - kio * 3
            fch = jnp.zeros((H, W), jnp.float32)
            for cc in range(c):
                fch = fch + jnp.where(ch_o == cc, fb[cc], 0.0)
            p_ref[2:2 + H, 2:2 + W] = fch
            R = jnp.zeros((H + 2, W + 2), jnp.float32)
            for a in range(3):
                for d in range(3):
                    hit = jnp.logical_and(kio == a, kjo == d)
                    R = R + jnp.where(hit, p_ref[a:a + H + 2, d:d + W + 2],
                                      0.0)
            for ki in range(3):
                for kj in range(3):
                    T = R[2 - ki:2 - ki + H, 2 - kj:2 - kj + W]
                    t = (fb * T[None, :, :]) * c_o
                    t = t / tot
                    valid = ((oy >= ki) & (oy <= Hh - 1 + ki) &
                             (ox >= kj) & (ox <= Ww - 1 + kj))
                    accb = accb + jnp.where(valid[None, :, :],
                                            jnp.floor(t), 0.0)
        outs.append(accb)
    o_ref[...] = jnp.stack(outs, axis=0)


@jax.jit
def kernel(feature):
    B, C, H, W = feature.shape
    fq = jnp.pad(feature, ((0, 0), (0, 0), (0, 2), (0, 34)))
    sel = pl.pallas_call(
        _select_kernel,
        grid=(_GRP,),
        in_specs=[pl.BlockSpec((B, _CPG, H + 2, W + 34),
                               lambda g: (0, g, 0, 0))],
        out_specs=pl.BlockSpec((1, 1, 8), lambda g: (g, 0, 0)),
        out_shape=jax.ShapeDtypeStruct((_GRP, 1, 8), jnp.int32),
    )(fq)
    return pl.pallas_call(
        _combine_kernel,
        grid=(_GRP,),
        in_specs=[pl.BlockSpec((B, _CPG, H + 2, W + 34),
                               lambda g: (0, g, 0, 0)),
                  pl.BlockSpec((1, 1, 8), lambda g: (g, 0, 0))],
        out_specs=pl.BlockSpec((B, _CPG, H, W), lambda g: (0, g, 0, 0)),
        out_shape=jax.ShapeDtypeStruct((B, C, H, W), jnp.float32),
        scratch_shapes=[pltpu.VMEM((H + 4, W + 4), jnp.float32)],
    )(fq, sel)
